# baseline (speedup 1.0000x reference)
import functools

import jax
import jax.numpy as jnp
from jax import lax
from jax.experimental import pallas as pl
from jax.experimental.pallas import tpu as pltpu
from jax.experimental.pallas import tpu_sc as plsc

_NC = 2
_NS = 16
_D = 256
_DH = _D // _NC
_NUM_HE = 10000
_IW = 125
_NBUF = 2
_CNTW = 8


def _seg_sum(src_lo, src_hi, gidx, sidx, n_rows, with_count):
    n_chunks_total = gidx.shape[0]
    n_chunks = n_chunks_total // _NS
    nh = n_chunks // 2
    rpt = n_rows // _NS

    mesh = plsc.VectorSubcoreMesh(core_axis_name="c", subcore_axis_name="s")
    out_type = [
        jax.ShapeDtypeStruct((n_rows, _DH), jnp.float32),
        jax.ShapeDtypeStruct((n_rows, _DH), jnp.float32),
    ]
    scratch = [
        pltpu.VMEM((nh, _IW), jnp.int32),
        pltpu.VMEM((nh, _IW), jnp.int32),
    ]
    scratch += [pltpu.VMEM((_IW, _DH), jnp.float32)
                for _ in range(_NBUF)]
    scratch += [pltpu.VMEM_SHARED((n_rows, _DH), jnp.float32)]
    scratch += [pltpu.SemaphoreType.DMA for _ in range(2 * _NBUF)]
    if with_count:
        out_type.append(jax.ShapeDtypeStruct((n_rows, _CNTW), jnp.float32))
        scratch += [
            pltpu.VMEM((_IW, _CNTW), jnp.float32),
            pltpu.VMEM_SHARED((n_rows, _CNTW), jnp.float32),
        ]

    @functools.partial(pl.kernel, out_type=out_type, mesh=mesh,
                       scratch_types=scratch,
                       compiler_params=pltpu.CompilerParams(
                           use_tc_tiling_on_sc=False,
                           internal_scratch_in_bytes=2 * 1024 * 1024))
    def body(zd_hbm, zc_hbm, on_hbm, lo_hbm, hi_hbm, gi_hbm, si_hbm, *rest):
        n_out = 3 if with_count else 2
        outs = rest[:n_out]
        out_lo, out_hi = outs[0], outs[1]
        gi_v, si_v = rest[n_out:n_out + 2]
        bufs = rest[n_out + 2:n_out + 2 + _NBUF]
        table = rest[n_out + 2 + _NBUF]
        gsems = rest[n_out + 3 + _NBUF:n_out + 3 + 2 * _NBUF]
        ssems = rest[n_out + 3 + 2 * _NBUF:n_out + 3 + 3 * _NBUF]
        if with_count:
            out_cnt = outs[2]
            ones_v, ctable = rest[n_out + 3 + 3 * _NBUF:]
        c = lax.axis_index("c")
        s = lax.axis_index("s")
        r0 = s * rpt

        def g_start(k, b):
            @pl.when(c == 0)
            def _():
                pltpu.async_copy(lo_hbm.at[gi_v.at[k]], bufs[b], gsems[b])

            @pl.when(c == 1)
            def _():
                pltpu.async_copy(hi_hbm.at[gi_v.at[k]], bufs[b], gsems[b])

        def g_wait(k, b):
            @pl.when(c == 0)
            def _():
                pltpu.make_async_copy(lo_hbm.at[gi_v.at[k]], bufs[b],
                                      gsems[b]).wait()

            @pl.when(c == 1)
            def _():
                pltpu.make_async_copy(hi_hbm.at[gi_v.at[k]], bufs[b],
                                      gsems[b]).wait()

        def s_start(k, b):
            pltpu.async_copy(bufs[b], table.at[si_v.at[k]], ssems[b],
                             add=True)

        def s_wait(k, b):
            pltpu.make_async_copy(bufs[b], table.at[si_v.at[k]],
                                  ssems[b]).wait()

        pltpu.sync_copy(zd_hbm.at[pl.ds(r0, rpt)], table.at[pl.ds(r0, rpt)])
        if with_count:
            @pl.when(c == 0)
            def _():
                pltpu.sync_copy(zc_hbm.at[pl.ds(r0, rpt)],
                                ctable.at[pl.ds(r0, rpt)])
                pltpu.sync_copy(on_hbm, ones_v)
        plsc.subcore_barrier()

        def cscat(k):
            if with_count:
                @pl.when(c == 0)
                def _():
                    pltpu.sync_copy(ones_v, ctable.at[si_v.at[k]], add=True)

        for h in range(2):
            pltpu.sync_copy(gi_hbm.at[pl.ds(s * n_chunks + h * nh, nh)], gi_v)
            pltpu.sync_copy(si_hbm.at[pl.ds(s * n_chunks + h * nh, nh)], si_v)
            g_start(0, 0)

            def step(j, carry):
                k0 = 2 * j
                k1 = k0 + 1
                g_start(k1, 1)
                g_wait(k0, 0)
                pltpu.sync_copy(bufs[0], table.at[si_v.at[k0]], add=True)
                cscat(k0)
                g_start(k0 + 2, 0)
                g_wait(k1, 1)
                pltpu.sync_copy(bufs[1], table.at[si_v.at[k1]], add=True)
                cscat(k1)
                return carry
            lax.fori_loop(0, nh // 2 - 1, step, 0)
            g_start(nh - 1, 1)
            g_wait(nh - 2, 0)
            pltpu.sync_copy(bufs[0], table.at[si_v.at[nh - 2]], add=True)
            cscat(nh - 2)
            g_wait(nh - 1, 1)
            pltpu.sync_copy(bufs[1], table.at[si_v.at[nh - 1]], add=True)
            cscat(nh - 1)
        plsc.subcore_barrier()

        @pl.when(c == 0)
        def _():
            pltpu.sync_copy(table.at[pl.ds(r0, rpt)], out_lo.at[pl.ds(r0, rpt)])
            if with_count:
                pltpu.sync_copy(ctable.at[pl.ds(r0, rpt)],
                                out_cnt.at[pl.ds(r0, rpt)])

        @pl.when(c == 1)
        def _():
            pltpu.sync_copy(table.at[pl.ds(r0, rpt)], out_hi.at[pl.ds(r0, rpt)])

    zeros_d = jnp.zeros((n_rows, _DH), jnp.float32)
    zeros_c = jnp.zeros((n_rows, _CNTW), jnp.float32)
    ones_r = jnp.ones((_IW, _CNTW), jnp.float32)
    return body(zeros_d, zeros_c, ones_r, src_lo, src_hi, gidx, sidx)


def _mlp(in_lo, in_hi, cnt, w1t_lo, w1t_hi, b1, w2t, b2, split_out):
    n = in_lo.shape[0]
    br = 1000
    grid = (n // br,)
    row_spec = pl.BlockSpec((br, _DH), lambda i: (i, 0))
    full = lambda shape: pl.BlockSpec(shape, lambda i: (0, 0))

    def body(*refs):
        if cnt is not None:
            lo_ref, hi_ref, cnt_ref, w1l, w1h, b1r, w2r, b2r = refs[:8]
            outs = refs[8:]
        else:
            lo_ref, hi_ref, w1l, w1h, b1r, w2r, b2r = refs[:7]
            outs = refs[7:]
        a_lo = lo_ref[...]
        a_hi = hi_ref[...]
        if cnt is not None:
            inv = 1.0 / jnp.maximum(cnt_ref[...][:, 0:1], 1.0)
            a_lo = a_lo * inv
            a_hi = a_hi * inv
        h = jnp.dot(a_lo, w1l[...], preferred_element_type=jnp.float32)
        h += jnp.dot(a_hi, w1h[...], preferred_element_type=jnp.float32)
        h = jnp.maximum(h + b1r[...], 0.0)
        o = jnp.dot(h, w2r[...], preferred_element_type=jnp.float32) + b2r[...]
        if split_out:
            outs[0][...] = o[:, :_DH]
            outs[1][...] = o[:, _DH:]
        else:
            outs[0][...] = o

    in_specs = [row_spec, row_spec]
    args = [in_lo, in_hi]
    if cnt is not None:
        in_specs.append(pl.BlockSpec((br, _CNTW), lambda i: (i, 0)))
        args.append(cnt)
    in_specs += [full((_DH, _D)), full((_DH, _D)), full((1, _D)),
                 full((_D, _D)), full((1, _D))]
    args += [w1t_lo, w1t_hi, b1.reshape(1, _D), w2t, b2.reshape(1, _D)]
    if split_out:
        out_shape = [jax.ShapeDtypeStruct((n, _DH), jnp.float32),
                     jax.ShapeDtypeStruct((n, _DH), jnp.float32)]
        out_specs = [row_spec, row_spec]
    else:
        out_shape = jax.ShapeDtypeStruct((n, _D), jnp.float32)
        out_specs = pl.BlockSpec((br, _D), lambda i: (i, 0))
    return pl.pallas_call(
        body, grid=grid, in_specs=in_specs, out_specs=out_specs,
        out_shape=out_shape)(*args)


def kernel(x, hyperedge_index, phi_w1, phi_b1, phi_w2, phi_b2,
           rho_w1, rho_b1, rho_w2, rho_b2):
    n_nodes = x.shape[0]
    node_idx = hyperedge_index[0].astype(jnp.int32).reshape(-1, _IW)
    he_idx = hyperedge_index[1].astype(jnp.int32).reshape(-1, _IW)

    x_lo = x[:, :_DH]
    x_hi = x[:, _DH:]

    he_lo, he_hi, he_cnt = _seg_sum(x_lo, x_hi, node_idx, he_idx,
                                    _NUM_HE, with_count=True)
    feat_lo, feat_hi = _mlp(he_lo, he_hi, he_cnt,
                            phi_w1.T[:_DH], phi_w1.T[_DH:], phi_b1,
                            phi_w2.T, phi_b2, split_out=True)
    sig_lo, sig_hi = _seg_sum(feat_lo, feat_hi, he_idx, node_idx,
                              n_nodes, with_count=False)
    out = _mlp(sig_lo, sig_hi, None,
               rho_w1.T[:_DH], rho_w1.T[_DH:], rho_b1,
               rho_w2.T, rho_b2, split_out=False)
    return out


# speedup vs baseline: 1.0247x; 1.0247x over previous
import functools

import jax
import jax.numpy as jnp
from jax import lax
from jax.experimental import pallas as pl
from jax.experimental.pallas import tpu as pltpu
from jax.experimental.pallas import tpu_sc as plsc

_NC = 2
_NS = 16
_D = 256
_DH = _D // _NC
_NUM_HE = 10000
_IW = 125
_NBUF = 2
_CNTW = 8


def _seg_sum(src_lo, src_hi, gidx, sidx, n_rows, with_count):
    n_chunks_total = gidx.shape[0]
    n_chunks = n_chunks_total // _NS
    nh = n_chunks // 2
    rpt = n_rows // _NS

    mesh = plsc.VectorSubcoreMesh(core_axis_name="c", subcore_axis_name="s")
    out_type = [
        jax.ShapeDtypeStruct((n_rows, _DH), jnp.float32),
        jax.ShapeDtypeStruct((n_rows, _DH), jnp.float32),
    ]
    scratch = [
        pltpu.VMEM((nh, _IW), jnp.int32),
        pltpu.VMEM((nh, _IW), jnp.int32),
    ]
    scratch += [pltpu.VMEM((_IW, _DH), jnp.float32)
                for _ in range(_NBUF)]
    scratch += [pltpu.VMEM_SHARED((n_rows, _DH), jnp.float32)]
    scratch += [pltpu.SemaphoreType.DMA for _ in range(2 * _NBUF)]
    if with_count:
        out_type.append(jax.ShapeDtypeStruct((n_rows, _CNTW), jnp.float32))
        scratch += [
            pltpu.VMEM((_IW, _CNTW), jnp.float32),
            pltpu.VMEM_SHARED((n_rows, _CNTW), jnp.float32),
            pltpu.SemaphoreType.DMA,
            pltpu.SemaphoreType.DMA,
        ]

    @functools.partial(pl.kernel, out_type=out_type, mesh=mesh,
                       scratch_types=scratch,
                       compiler_params=pltpu.CompilerParams(
                           use_tc_tiling_on_sc=False,
                           internal_scratch_in_bytes=2 * 1024 * 1024))
    def body(zd_hbm, zc_hbm, on_hbm, lo_hbm, hi_hbm, gi_hbm, si_hbm, *rest):
        n_out = 3 if with_count else 2
        outs = rest[:n_out]
        out_lo, out_hi = outs[0], outs[1]
        gi_v, si_v = rest[n_out:n_out + 2]
        bufs = rest[n_out + 2:n_out + 2 + _NBUF]
        table = rest[n_out + 2 + _NBUF]
        gsems = rest[n_out + 3 + _NBUF:n_out + 3 + 2 * _NBUF]
        ssems = rest[n_out + 3 + 2 * _NBUF:n_out + 3 + 3 * _NBUF]
        if with_count:
            out_cnt = outs[2]
            ones_v, ctable, csem0, csem1 = rest[n_out + 3 + 3 * _NBUF:]
            csems = (csem0, csem1)
        c = lax.axis_index("c")
        s = lax.axis_index("s")
        r0 = s * rpt

        def g_start(k, b):
            @pl.when(c == 0)
            def _():
                pltpu.async_copy(lo_hbm.at[gi_v.at[k]], bufs[b], gsems[b])

            @pl.when(c == 1)
            def _():
                pltpu.async_copy(hi_hbm.at[gi_v.at[k]], bufs[b], gsems[b])

        def g_wait(k, b):
            @pl.when(c == 0)
            def _():
                pltpu.make_async_copy(lo_hbm.at[gi_v.at[k]], bufs[b],
                                      gsems[b]).wait()

            @pl.when(c == 1)
            def _():
                pltpu.make_async_copy(hi_hbm.at[gi_v.at[k]], bufs[b],
                                      gsems[b]).wait()

        def s_start(k, b):
            pltpu.async_copy(bufs[b], table.at[si_v.at[k]], ssems[b],
                             add=True)

        def s_wait(k, b):
            pltpu.make_async_copy(bufs[b], table.at[si_v.at[k]],
                                  ssems[b]).wait()

        pltpu.sync_copy(zd_hbm.at[pl.ds(r0, rpt)], table.at[pl.ds(r0, rpt)])
        if with_count:
            @pl.when(c == 0)
            def _():
                pltpu.sync_copy(zc_hbm.at[pl.ds(r0, rpt)],
                                ctable.at[pl.ds(r0, rpt)])
                pltpu.sync_copy(on_hbm, ones_v)
        plsc.subcore_barrier()

        def cscat(k, b):
            if with_count:
                @pl.when(c == 0)
                def _():
                    @pl.when(k >= 2)
                    def _():
                        pltpu.make_async_copy(
                            ones_v, ctable.at[si_v.at[k - 2]],
                            csems[b]).wait()
                    pltpu.async_copy(ones_v, ctable.at[si_v.at[k]], csems[b],
                                     add=True)

        def cdrain(k, b):
            if with_count:
                @pl.when(c == 0)
                def _():
                    pltpu.make_async_copy(ones_v, ctable.at[si_v.at[k]],
                                          csems[b]).wait()

        for h in range(2):
            pltpu.sync_copy(gi_hbm.at[pl.ds(s * n_chunks + h * nh, nh)], gi_v)
            pltpu.sync_copy(si_hbm.at[pl.ds(s * n_chunks + h * nh, nh)], si_v)
            g_start(0, 0)

            def step(j, carry):
                k0 = 2 * j
                k1 = k0 + 1
                g_start(k1, 1)
                g_wait(k0, 0)
                pltpu.sync_copy(bufs[0], table.at[si_v.at[k0]], add=True)
                cscat(k0, 0)
                g_start(k0 + 2, 0)
                g_wait(k1, 1)
                pltpu.sync_copy(bufs[1], table.at[si_v.at[k1]], add=True)
                cscat(k1, 1)
                return carry
            lax.fori_loop(0, nh // 2 - 1, step, 0)
            g_start(nh - 1, 1)
            g_wait(nh - 2, 0)
            pltpu.sync_copy(bufs[0], table.at[si_v.at[nh - 2]], add=True)
            cscat(nh - 2, 0)
            g_wait(nh - 1, 1)
            pltpu.sync_copy(bufs[1], table.at[si_v.at[nh - 1]], add=True)
            cscat(nh - 1, 1)
            cdrain(nh - 2, 0)
            cdrain(nh - 1, 1)
        plsc.subcore_barrier()

        @pl.when(c == 0)
        def _():
            pltpu.sync_copy(table.at[pl.ds(r0, rpt)], out_lo.at[pl.ds(r0, rpt)])
            if with_count:
                pltpu.sync_copy(ctable.at[pl.ds(r0, rpt)],
                                out_cnt.at[pl.ds(r0, rpt)])

        @pl.when(c == 1)
        def _():
            pltpu.sync_copy(table.at[pl.ds(r0, rpt)], out_hi.at[pl.ds(r0, rpt)])

    zeros_d = jnp.zeros((n_rows, _DH), jnp.float32)
    zeros_c = jnp.zeros((n_rows, _CNTW), jnp.float32)
    ones_r = jnp.ones((_IW, _CNTW), jnp.float32)
    return body(zeros_d, zeros_c, ones_r, src_lo, src_hi, gidx, sidx)


def _mlp(in_lo, in_hi, cnt, w1t_lo, w1t_hi, b1, w2t, b2, split_out):
    n = in_lo.shape[0]
    br = 2000
    grid = (n // br,)
    row_spec = pl.BlockSpec((br, _DH), lambda i: (i, 0))
    full = lambda shape: pl.BlockSpec(shape, lambda i: (0, 0))

    def body(*refs):
        if cnt is not None:
            lo_ref, hi_ref, cnt_ref, w1l, w1h, b1r, w2r, b2r = refs[:8]
            outs = refs[8:]
        else:
            lo_ref, hi_ref, w1l, w1h, b1r, w2r, b2r = refs[:7]
            outs = refs[7:]
        a_lo = lo_ref[...]
        a_hi = hi_ref[...]
        if cnt is not None:
            inv = 1.0 / jnp.maximum(cnt_ref[...][:, 0:1], 1.0)
            a_lo = a_lo * inv
            a_hi = a_hi * inv
        h = jnp.dot(a_lo, w1l[...], preferred_element_type=jnp.float32)
        h += jnp.dot(a_hi, w1h[...], preferred_element_type=jnp.float32)
        h = jnp.maximum(h + b1r[...], 0.0)
        o = jnp.dot(h, w2r[...], preferred_element_type=jnp.float32) + b2r[...]
        if split_out:
            outs[0][...] = o[:, :_DH]
            outs[1][...] = o[:, _DH:]
        else:
            outs[0][...] = o

    in_specs = [row_spec, row_spec]
    args = [in_lo, in_hi]
    if cnt is not None:
        in_specs.append(pl.BlockSpec((br, _CNTW), lambda i: (i, 0)))
        args.append(cnt)
    in_specs += [full((_DH, _D)), full((_DH, _D)), full((1, _D)),
                 full((_D, _D)), full((1, _D))]
    args += [w1t_lo, w1t_hi, b1.reshape(1, _D), w2t, b2.reshape(1, _D)]
    if split_out:
        out_shape = [jax.ShapeDtypeStruct((n, _DH), jnp.float32),
                     jax.ShapeDtypeStruct((n, _DH), jnp.float32)]
        out_specs = [row_spec, row_spec]
    else:
        out_shape = jax.ShapeDtypeStruct((n, _D), jnp.float32)
        out_specs = pl.BlockSpec((br, _D), lambda i: (i, 0))
    return pl.pallas_call(
        body, grid=grid, in_specs=in_specs, out_specs=out_specs,
        out_shape=out_shape)(*args)


def kernel(x, hyperedge_index, phi_w1, phi_b1, phi_w2, phi_b2,
           rho_w1, rho_b1, rho_w2, rho_b2):
    n_nodes = x.shape[0]
    node_idx = hyperedge_index[0].astype(jnp.int32).reshape(-1, _IW)
    he_idx = hyperedge_index[1].astype(jnp.int32).reshape(-1, _IW)

    x_lo = x[:, :_DH]
    x_hi = x[:, _DH:]

    he_lo, he_hi, he_cnt = _seg_sum(x_lo, x_hi, node_idx, he_idx,
                                    _NUM_HE, with_count=True)
    feat_lo, feat_hi = _mlp(he_lo, he_hi, he_cnt,
                            phi_w1.T[:_DH], phi_w1.T[_DH:], phi_b1,
                            phi_w2.T, phi_b2, split_out=True)
    sig_lo, sig_hi = _seg_sum(feat_lo, feat_hi, he_idx, node_idx,
                              n_nodes, with_count=False)
    out = _mlp(sig_lo, sig_hi, None,
               rho_w1.T[:_DH], rho_w1.T[_DH:], rho_b1,
               rho_w2.T, rho_b2, split_out=False)
    return out


# speedup vs baseline: 1.0389x; 1.0138x over previous
import functools

import jax
import jax.numpy as jnp
from jax import lax
from jax.experimental import pallas as pl
from jax.experimental.pallas import tpu as pltpu
from jax.experimental.pallas import tpu_sc as plsc

_NC = 2
_NS = 16
_D = 256
_DH = _D // _NC
_NUM_HE = 10000
_IW = 125
_NBUF = 2
_CNTW = 8


def _seg_sum(src_lo, src_hi, gidx0, gidx1, sidx, n_rows, with_count):
    n_chunks_total = gidx0.shape[0]
    n_chunks = n_chunks_total // _NS
    nh = n_chunks // 2
    rpt = n_rows // _NS

    mesh = plsc.VectorSubcoreMesh(core_axis_name="c", subcore_axis_name="s")
    out_type = [
        jax.ShapeDtypeStruct((n_rows, _DH), jnp.float32),
        jax.ShapeDtypeStruct((n_rows, _DH), jnp.float32),
    ]
    scratch = [
        pltpu.VMEM((nh, _IW), jnp.int32),
        pltpu.VMEM((nh, _IW), jnp.int32),
    ]
    scratch += [pltpu.VMEM((_IW, _DH), jnp.float32)
                for _ in range(_NBUF)]
    scratch += [pltpu.VMEM_SHARED((n_rows, _DH), jnp.float32)]
    scratch += [pltpu.SemaphoreType.DMA for _ in range(2 * _NBUF)]
    if with_count:
        out_type.append(jax.ShapeDtypeStruct((n_rows, _CNTW), jnp.float32))
        scratch += [
            pltpu.VMEM((_IW, _CNTW), jnp.float32),
            pltpu.VMEM_SHARED((n_rows, _CNTW), jnp.float32),
            pltpu.SemaphoreType.DMA,
            pltpu.SemaphoreType.DMA,
        ]

    @functools.partial(pl.kernel, out_type=out_type, mesh=mesh,
                       scratch_types=scratch,
                       compiler_params=pltpu.CompilerParams(
                           use_tc_tiling_on_sc=False,
                           internal_scratch_in_bytes=2 * 1024 * 1024))
    def body(zd_hbm, zc_hbm, on_hbm, lo_hbm, hi_hbm, gi0_hbm, gi1_hbm,
             si_hbm, *rest):
        n_out = 3 if with_count else 2
        outs = rest[:n_out]
        out_lo, out_hi = outs[0], outs[1]
        gi_v, si_v = rest[n_out:n_out + 2]
        bufs = rest[n_out + 2:n_out + 2 + _NBUF]
        table = rest[n_out + 2 + _NBUF]
        gsems = rest[n_out + 3 + _NBUF:n_out + 3 + 2 * _NBUF]
        ssems = rest[n_out + 3 + 2 * _NBUF:n_out + 3 + 3 * _NBUF]
        if with_count:
            out_cnt = outs[2]
            ones_v, ctable, csem0, csem1 = rest[n_out + 3 + 3 * _NBUF:]
            csems = (csem0, csem1)
        c = lax.axis_index("c")
        s = lax.axis_index("s")
        r0 = s * rpt

        def g_start(k, b):
            @pl.when(c == 0)
            def _():
                pltpu.async_copy(lo_hbm.at[gi_v.at[k]], bufs[b], gsems[b])

            @pl.when(c == 1)
            def _():
                pltpu.async_copy(hi_hbm.at[gi_v.at[k]], bufs[b], gsems[b])

        def g_wait(k, b):
            @pl.when(c == 0)
            def _():
                pltpu.make_async_copy(lo_hbm.at[gi_v.at[k]], bufs[b],
                                      gsems[b]).wait()

            @pl.when(c == 1)
            def _():
                pltpu.make_async_copy(hi_hbm.at[gi_v.at[k]], bufs[b],
                                      gsems[b]).wait()

        def s_start(k, b):
            pltpu.async_copy(bufs[b], table.at[si_v.at[k]], ssems[b],
                             add=True)

        def s_wait(k, b):
            pltpu.make_async_copy(bufs[b], table.at[si_v.at[k]],
                                  ssems[b]).wait()

        pltpu.sync_copy(zd_hbm.at[pl.ds(r0, rpt)], table.at[pl.ds(r0, rpt)])
        if with_count:
            @pl.when(c == 0)
            def _():
                pltpu.sync_copy(zc_hbm.at[pl.ds(r0, rpt)],
                                ctable.at[pl.ds(r0, rpt)])
                pltpu.sync_copy(on_hbm, ones_v)
        plsc.subcore_barrier()

        def cscat(k, b):
            if with_count:
                @pl.when(c == 0)
                def _():
                    @pl.when(k >= 2)
                    def _():
                        pltpu.make_async_copy(
                            ones_v, ctable.at[si_v.at[k - 2]],
                            csems[b]).wait()
                    pltpu.async_copy(ones_v, ctable.at[si_v.at[k]], csems[b],
                                     add=True)

        def cdrain(k, b):
            if with_count:
                @pl.when(c == 0)
                def _():
                    pltpu.make_async_copy(ones_v, ctable.at[si_v.at[k]],
                                          csems[b]).wait()

        for h in range(2):
            @pl.when(c == 0)
            def _():
                pltpu.sync_copy(gi0_hbm.at[pl.ds(s * n_chunks + h * nh, nh)],
                                gi_v)

            @pl.when(c == 1)
            def _():
                pltpu.sync_copy(gi1_hbm.at[pl.ds(s * n_chunks + h * nh, nh)],
                                gi_v)
            pltpu.sync_copy(si_hbm.at[pl.ds(s * n_chunks + h * nh, nh)], si_v)
            g_start(0, 0)

            def step(j, carry):
                k0 = 2 * j
                k1 = k0 + 1
                g_start(k1, 1)
                g_wait(k0, 0)
                pltpu.sync_copy(bufs[0], table.at[si_v.at[k0]], add=True)
                cscat(k0, 0)
                g_start(k0 + 2, 0)
                g_wait(k1, 1)
                pltpu.sync_copy(bufs[1], table.at[si_v.at[k1]], add=True)
                cscat(k1, 1)
                return carry
            lax.fori_loop(0, nh // 2 - 1, step, 0)
            g_start(nh - 1, 1)
            g_wait(nh - 2, 0)
            pltpu.sync_copy(bufs[0], table.at[si_v.at[nh - 2]], add=True)
            cscat(nh - 2, 0)
            g_wait(nh - 1, 1)
            pltpu.sync_copy(bufs[1], table.at[si_v.at[nh - 1]], add=True)
            cscat(nh - 1, 1)
            cdrain(nh - 2, 0)
            cdrain(nh - 1, 1)
        plsc.subcore_barrier()

        @pl.when(c == 0)
        def _():
            pltpu.sync_copy(table.at[pl.ds(r0, rpt)], out_lo.at[pl.ds(r0, rpt)])
            if with_count:
                pltpu.sync_copy(ctable.at[pl.ds(r0, rpt)],
                                out_cnt.at[pl.ds(r0, rpt)])

        @pl.when(c == 1)
        def _():
            pltpu.sync_copy(table.at[pl.ds(r0, rpt)], out_hi.at[pl.ds(r0, rpt)])

    zeros_d = jnp.zeros((n_rows, _DH), jnp.float32)
    zeros_c = jnp.zeros((n_rows, _CNTW), jnp.float32)
    ones_r = jnp.ones((_IW, _CNTW), jnp.float32)
    return body(zeros_d, zeros_c, ones_r, src_lo, src_hi, gidx0, gidx1, sidx)


def _mlp(in_lo, in_hi, cnt, w1t_lo, w1t_hi, b1, w2t, b2, split_out):
    n = in_lo.shape[0]
    br = 2000
    grid = (n // br,)
    row_spec = pl.BlockSpec((br, _DH), lambda i: (i, 0))
    full = lambda shape: pl.BlockSpec(shape, lambda i: (0, 0))

    def body(*refs):
        if cnt is not None:
            lo_ref, hi_ref, cnt_ref, w1l, w1h, b1r, w2r, b2r = refs[:8]
            outs = refs[8:]
        else:
            lo_ref, hi_ref, w1l, w1h, b1r, w2r, b2r = refs[:7]
            outs = refs[7:]
        a_lo = lo_ref[...]
        a_hi = hi_ref[...]
        if cnt is not None:
            inv = 1.0 / jnp.maximum(cnt_ref[...][:, 0:1], 1.0)
            a_lo = a_lo * inv
            a_hi = a_hi * inv
        h = jnp.dot(a_lo, w1l[...], preferred_element_type=jnp.float32)
        h += jnp.dot(a_hi, w1h[...], preferred_element_type=jnp.float32)
        h = jnp.maximum(h + b1r[...], 0.0)
        o = jnp.dot(h, w2r[...], preferred_element_type=jnp.float32) + b2r[...]
        if split_out:
            outs[0][...] = o[:, :_DH]
            outs[1][...] = o[:, _DH:]
        else:
            outs[0][...] = o

    in_specs = [row_spec, row_spec]
    args = [in_lo, in_hi]
    if cnt is not None:
        in_specs.append(pl.BlockSpec((br, _CNTW), lambda i: (i, 0)))
        args.append(cnt)
    in_specs += [full((_DH, _D)), full((_DH, _D)), full((1, _D)),
                 full((_D, _D)), full((1, _D))]
    args += [w1t_lo, w1t_hi, b1.reshape(1, _D), w2t, b2.reshape(1, _D)]
    if split_out:
        out_shape = [jax.ShapeDtypeStruct((n, _DH), jnp.float32),
                     jax.ShapeDtypeStruct((n, _DH), jnp.float32)]
        out_specs = [row_spec, row_spec]
    else:
        out_shape = jax.ShapeDtypeStruct((n, _D), jnp.float32)
        out_specs = pl.BlockSpec((br, _D), lambda i: (i, 0))
    return pl.pallas_call(
        body, grid=grid, in_specs=in_specs, out_specs=out_specs,
        out_shape=out_shape)(*args)


def kernel(x, hyperedge_index, phi_w1, phi_b1, phi_w2, phi_b2,
           rho_w1, rho_b1, rho_w2, rho_b2):
    n_nodes = x.shape[0]
    node_idx = hyperedge_index[0].astype(jnp.int32)
    he_idx = hyperedge_index[1].astype(jnp.int32).reshape(-1, _IW)

    x2 = x.reshape(-1, _DH)
    ni0 = (node_idx * 2).reshape(-1, _IW)
    ni1 = (node_idx * 2 + 1).reshape(-1, _IW)

    he_lo, he_hi, he_cnt = _seg_sum(x2, x2, ni0, ni1, he_idx,
                                    _NUM_HE, with_count=True)
    feat_lo, feat_hi = _mlp(he_lo, he_hi, he_cnt,
                            phi_w1.T[:_DH], phi_w1.T[_DH:], phi_b1,
                            phi_w2.T, phi_b2, split_out=True)
    node_idx2 = node_idx.reshape(-1, _IW)
    sig_lo, sig_hi = _seg_sum(feat_lo, feat_hi, he_idx, he_idx, node_idx2,
                              n_nodes, with_count=False)
    out = _mlp(sig_lo, sig_hi, None,
               rho_w1.T[:_DH], rho_w1.T[_DH:], rho_b1,
               rho_w2.T, rho_b2, split_out=False)
    return out
